# baseline (speedup 1.0000x reference)
import jax
import jax.numpy as jnp
from jax.experimental import pallas as pl

N = 100000
STATIC_DIM = 128
DYNAMIC_DIM = 128
BLOCK_N = 2000


def _concat_block(a_ref, b_ref, o_ref):
    o_ref[:, :STATIC_DIM] = a_ref[:]
    o_ref[:, STATIC_DIM:] = b_ref[:]


def kernel(static_emb, dynamic_emb):
    grid = (N // BLOCK_N,)
    return pl.pallas_call(
        _concat_block,
        grid=grid,
        in_specs=[
            pl.BlockSpec((BLOCK_N, STATIC_DIM), lambda i: (i, 0)),
            pl.BlockSpec((BLOCK_N, DYNAMIC_DIM), lambda i: (i, 0)),
        ],
        out_specs=pl.BlockSpec((BLOCK_N, STATIC_DIM + DYNAMIC_DIM),
                               lambda i: (i, 0)),
        out_shape=jax.ShapeDtypeStruct((N, STATIC_DIM + DYNAMIC_DIM),
                                       jnp.float32),
    )(static_emb, dynamic_emb)
